# baseline (device time: 93352 ns/iter reference)
import os

import jax
import jax.numpy as jnp
from jax import lax
from jax.experimental import pallas as pl
from jax.experimental.pallas import tpu as pltpu

N_DEV = 32
_VARIANT = os.environ.get("KERNEL_VARIANT", "full")


def kernel(x, Win0, Wout0, Win1, Wout1, Win2, Wout2):
    b, d_per = x.shape
    _, h_dim = Win0.shape
    assert b == 512 and h_dim == 512

    def body(x_ref, win0_ref, wout0_ref, win1_ref, wout1_ref, win2_ref,
             wout2_ref, out_ref,
             pbuf, rbx, sbuf, rby, tbuf, rbz, hbuf,
             sx_s, sx_r, sy_s, sy_r, sz_s, sz_r,
             gx_s, gx_r, gy_s, gy_r, gz_s, gz_r):
        me = lax.axis_index("i")
        zc = lax.div(me, 8)
        r = lax.rem(me, 8)
        yc = lax.div(r, 2)
        e = lax.rem(r, 2)
        p = lax.rem(yc, 2)
        xc = lax.rem(e + p, 2)

        x_partner = me + 1 - 2 * e

        def y_partner(yp):
            return zc * 8 + yp * 2 + lax.rem(e + p + lax.rem(yp, 2), 2)

        def z_partner(zp):
            return zp * 8 + r

        half = xc * 256
        quarter = half + yc * 64
        chunk_row = quarter + zc * 16

        def all_reduce_relu(partial_f32):
            if _VARIANT == "nocomm":
                hbuf[...] = jnp.maximum(partial_f32, 0.0).astype(jnp.bfloat16)
                return
            pbuf[...] = partial_f32.astype(jnp.bfloat16)

            dx = pltpu.make_async_remote_copy(
                src_ref=pbuf.at[pl.ds((1 - xc) * 256, 256), :],
                dst_ref=rbx.at[:, :],
                send_sem=sx_s.at[0],
                recv_sem=sx_r.at[0],
                device_id=(x_partner,),
                device_id_type=pl.DeviceIdType.MESH,
            )
            dx.start()
            dx.wait_recv()
            sbuf[...] = (pbuf[pl.ds(half, 256), :].astype(jnp.float32)
                         + rbx[...].astype(jnp.float32)).astype(jnp.bfloat16)
            dx.wait_send()

            sends = []
            for dy in range(1, 4):
                yp = lax.rem(yc + dy, 4)
                d = pltpu.make_async_remote_copy(
                    src_ref=sbuf.at[pl.ds(yp * 64, 64), :],
                    dst_ref=rby.at[yc],
                    send_sem=sy_s.at[yp],
                    recv_sem=sy_r.at[yc],
                    device_id=(y_partner(yp),),
                    device_id_type=pl.DeviceIdType.MESH,
                )
                d.start()
                sends.append(d)
            rby[pl.ds(yc, 1)] = sbuf[pl.ds(yc * 64, 64), :][None]
            for dy in range(1, 4):
                ys = lax.rem(yc - dy + 4, 4)
                pltpu.make_async_remote_copy(
                    src_ref=sbuf.at[pl.ds(0, 64), :],
                    dst_ref=rby.at[ys],
                    send_sem=sy_s.at[0],
                    recv_sem=sy_r.at[ys],
                    device_id=(me,),
                    device_id_type=pl.DeviceIdType.MESH,
                ).wait_recv()
            for d in sends:
                d.wait_send()
            tbuf[...] = jnp.sum(rby[...].astype(jnp.float32),
                                axis=0).astype(jnp.bfloat16)

            sends = []
            for dz in range(1, 4):
                zp = lax.rem(zc + dz, 4)
                d = pltpu.make_async_remote_copy(
                    src_ref=tbuf.at[pl.ds(zp * 16, 16), :],
                    dst_ref=rbz.at[zc],
                    send_sem=sz_s.at[zp],
                    recv_sem=sz_r.at[zc],
                    device_id=(z_partner(zp),),
                    device_id_type=pl.DeviceIdType.MESH,
                )
                d.start()
                sends.append(d)
            rbz[pl.ds(zc, 1)] = tbuf[pl.ds(zc * 16, 16), :][None]
            for dz in range(1, 4):
                zs = lax.rem(zc - dz + 4, 4)
                pltpu.make_async_remote_copy(
                    src_ref=tbuf.at[pl.ds(0, 16), :],
                    dst_ref=rbz.at[zs],
                    send_sem=sz_s.at[0],
                    recv_sem=sz_r.at[zs],
                    device_id=(me,),
                    device_id_type=pl.DeviceIdType.MESH,
                ).wait_recv()
            for d in sends:
                d.wait_send()
            s = jnp.sum(rbz[...].astype(jnp.float32), axis=0)
            s = jnp.maximum(s, 0.0).astype(jnp.bfloat16)

            hbuf[pl.ds(chunk_row, 16), :] = s
            sends = []
            for dz in range(1, 4):
                zp = lax.rem(zc + dz, 4)
                d = pltpu.make_async_remote_copy(
                    src_ref=hbuf.at[pl.ds(chunk_row, 16), :],
                    dst_ref=hbuf.at[pl.ds(chunk_row, 16), :],
                    send_sem=gz_s.at[zp],
                    recv_sem=gz_r.at[zc],
                    device_id=(z_partner(zp),),
                    device_id_type=pl.DeviceIdType.MESH,
                )
                d.start()
                sends.append(d)
            for dz in range(1, 4):
                zs = lax.rem(zc - dz + 4, 4)
                pltpu.make_async_remote_copy(
                    src_ref=hbuf.at[pl.ds(0, 16), :],
                    dst_ref=hbuf.at[pl.ds(quarter + zs * 16, 16), :],
                    send_sem=gz_s.at[0],
                    recv_sem=gz_r.at[zs],
                    device_id=(me,),
                    device_id_type=pl.DeviceIdType.MESH,
                ).wait_recv()
            for d in sends:
                d.wait_send()

            sends = []
            for dy in range(1, 4):
                yp = lax.rem(yc + dy, 4)
                d = pltpu.make_async_remote_copy(
                    src_ref=hbuf.at[pl.ds(quarter, 64), :],
                    dst_ref=hbuf.at[pl.ds(quarter, 64), :],
                    send_sem=gy_s.at[yp],
                    recv_sem=gy_r.at[yc],
                    device_id=(y_partner(yp),),
                    device_id_type=pl.DeviceIdType.MESH,
                )
                d.start()
                sends.append(d)
            for dy in range(1, 4):
                ys = lax.rem(yc - dy + 4, 4)
                pltpu.make_async_remote_copy(
                    src_ref=hbuf.at[pl.ds(0, 64), :],
                    dst_ref=hbuf.at[pl.ds(half + ys * 64, 64), :],
                    send_sem=gy_s.at[0],
                    recv_sem=gy_r.at[ys],
                    device_id=(me,),
                    device_id_type=pl.DeviceIdType.MESH,
                ).wait_recv()
            for d in sends:
                d.wait_send()

            dgx = pltpu.make_async_remote_copy(
                src_ref=hbuf.at[pl.ds(half, 256), :],
                dst_ref=hbuf.at[pl.ds(half, 256), :],
                send_sem=gx_s.at[0],
                recv_sem=gx_r.at[0],
                device_id=(x_partner,),
                device_id_type=pl.DeviceIdType.MESH,
            )
            dgx.start()
            pltpu.make_async_remote_copy(
                src_ref=hbuf.at[pl.ds(0, 256), :],
                dst_ref=hbuf.at[pl.ds((1 - xc) * 256, 256), :],
                send_sem=gx_s.at[0],
                recv_sem=gx_r.at[0],
                device_id=(me,),
                device_id_type=pl.DeviceIdType.MESH,
            ).wait_recv()
            dgx.wait_send()

        xb = x_ref[...].astype(jnp.bfloat16)
        xf = None
        for win_ref, wout_ref in ((win0_ref, wout0_ref),
                                  (win1_ref, wout1_ref),
                                  (win2_ref, wout2_ref)):
            wb = win_ref[...].astype(jnp.bfloat16)
            partial = jnp.dot(xb, wb, preferred_element_type=jnp.float32)
            all_reduce_relu(partial)
            wob = wout_ref[...].astype(jnp.bfloat16)
            xf = jnp.dot(hbuf[...], wob, preferred_element_type=jnp.float32)
            xb = xf.astype(jnp.bfloat16)
        out_ref[...] = xf

    return pl.pallas_call(
        body,
        out_shape=jax.ShapeDtypeStruct((b, d_per), jnp.float32),
        in_specs=[pl.BlockSpec(memory_space=pltpu.VMEM)] * 7,
        out_specs=pl.BlockSpec(memory_space=pltpu.VMEM),
        scratch_shapes=[
            pltpu.VMEM((b, h_dim), jnp.bfloat16),
            pltpu.VMEM((256, h_dim), jnp.bfloat16),
            pltpu.VMEM((256, h_dim), jnp.bfloat16),
            pltpu.VMEM((4, 64, h_dim), jnp.bfloat16),
            pltpu.VMEM((64, h_dim), jnp.bfloat16),
            pltpu.VMEM((4, 16, h_dim), jnp.bfloat16),
            pltpu.VMEM((b, h_dim), jnp.bfloat16),
            pltpu.SemaphoreType.DMA((1,)),
            pltpu.SemaphoreType.DMA((1,)),
            pltpu.SemaphoreType.DMA((4,)),
            pltpu.SemaphoreType.DMA((4,)),
            pltpu.SemaphoreType.DMA((4,)),
            pltpu.SemaphoreType.DMA((4,)),
            pltpu.SemaphoreType.DMA((1,)),
            pltpu.SemaphoreType.DMA((1,)),
            pltpu.SemaphoreType.DMA((4,)),
            pltpu.SemaphoreType.DMA((4,)),
            pltpu.SemaphoreType.DMA((4,)),
            pltpu.SemaphoreType.DMA((4,)),
        ],
    )(x, Win0, Wout0, Win1, Wout1, Win2, Wout2)


# device time: 72579 ns/iter; 1.2862x vs baseline; 1.2862x over previous
import os

import jax
import jax.numpy as jnp
from jax import lax
from jax.experimental import pallas as pl
from jax.experimental.pallas import tpu as pltpu

N_DEV = 32
GROUPS = [(1, 8), (9, 8), (17, 8), (25, 7)]
_VARIANT = os.environ.get("KERNEL_VARIANT", "full")


def kernel(x, Win0, Wout0, Win1, Wout1, Win2, Wout2):
    b, d_per = x.shape
    _, h_dim = Win0.shape
    chunk = b // N_DEV

    def body(x_ref, win0_ref, wout0_ref, win1_ref, wout1_ref, win2_ref,
             wout2_ref, out_ref, pbuf, qbuf, recv1, recv2, ss1, rs1, ss2, rs2):
        me = lax.axis_index("i")

        def rs_send(src_slice_fn, l):
            sends = []
            for off in range(1, N_DEV):
                tgt = lax.rem(me + off, N_DEV)
                woff = lax.rem(me - tgt + N_DEV, N_DEV)
                d = pltpu.make_async_remote_copy(
                    src_ref=src_slice_fn(off, tgt),
                    dst_ref=recv1.at[woff],
                    send_sem=ss1.at[off - 1],
                    recv_sem=rs1.at[woff],
                    device_id=(tgt,),
                    device_id_type=pl.DeviceIdType.MESH,
                )
                d.start()
                sends.append(d)
            return sends

        def reduce_relu(l):
            with jax.named_scope(f"reduce#l={l}"):
                if True:
                    for off in range(1, N_DEV):
                        pltpu.make_async_remote_copy(
                            src_ref=recv1.at[0],
                            dst_ref=recv1.at[off],
                            send_sem=ss1.at[0],
                            recv_sem=rs1.at[off],
                            device_id=(me,),
                            device_id_type=pl.DeviceIdType.MESH,
                        ).wait_recv()
                    acc = jnp.sum(recv1[...].astype(jnp.float32), axis=0)
                    return jnp.maximum(acc, 0.0).astype(jnp.bfloat16)
                acc = recv1[pl.ds(0, 1)].astype(jnp.float32)[0]
                for lo, n in GROUPS:
                    for off in range(lo, lo + n):
                        pltpu.make_async_remote_copy(
                            src_ref=recv1.at[0],
                            dst_ref=recv1.at[off],
                            send_sem=ss1.at[0],
                            recv_sem=rs1.at[off],
                            device_id=(me,),
                            device_id_type=pl.DeviceIdType.MESH,
                        ).wait_recv()
                    acc = acc + jnp.sum(
                        recv1[pl.ds(lo, n)].astype(jnp.float32), axis=0)
                return jnp.maximum(acc, 0.0).astype(jnp.bfloat16)

        def ag_send(l):
            sends = []
            for off in range(1, N_DEV):
                tgt = lax.rem(me + off, N_DEV)
                woff = lax.rem(me - tgt + N_DEV, N_DEV)
                d = pltpu.make_async_remote_copy(
                    src_ref=recv2.at[0],
                    dst_ref=recv2.at[woff],
                    send_sem=ss2.at[off - 1],
                    recv_sem=rs2.at[woff],
                    device_id=(tgt,),
                    device_id_type=pl.DeviceIdType.MESH,
                )
                d.start()
                sends.append(d)
            return sends

        def ag_wait_group(lo, n):
            for off in range(lo, lo + n):
                pltpu.make_async_remote_copy(
                    src_ref=recv2.at[0],
                    dst_ref=recv2.at[off],
                    send_sem=ss2.at[0],
                    recv_sem=rs2.at[off],
                    device_id=(me,),
                    device_id_type=pl.DeviceIdType.MESH,
                ).wait_recv()

        if _VARIANT == "nocomm":
            xb = x_ref[...].astype(jnp.bfloat16)
            xf = None
            for win_ref, wout_ref in ((win0_ref, wout0_ref),
                                      (win1_ref, wout1_ref),
                                      (win2_ref, wout2_ref)):
                p = jnp.dot(xb, win_ref[...].astype(jnp.bfloat16),
                            preferred_element_type=jnp.float32)
                h = jnp.maximum(p, 0.0).astype(jnp.bfloat16)
                xf = jnp.dot(h, wout_ref[...].astype(jnp.bfloat16),
                             preferred_element_type=jnp.float32)
                xb = xf.astype(jnp.bfloat16)
            out_ref[...] = xf
            return

        with jax.named_scope("matmul_in#l=0"):
            xb = x_ref[...].astype(jnp.bfloat16)
            partial = jnp.dot(xb, win0_ref[...].astype(jnp.bfloat16),
                              preferred_element_type=jnp.float32)
            pbuf[...] = partial.astype(jnp.bfloat16)
        with jax.named_scope("rs_send#l=0"):
            rs_sends = rs_send(
                lambda off, tgt: pbuf.at[pl.ds(tgt * chunk, chunk), :], 0)
            recv1[pl.ds(0, 1)] = pbuf[pl.ds(me * chunk, chunk), :][None]
        s = reduce_relu(0)
        for d in rs_sends:
            d.wait_send()

        if _VARIANT in ("rs0_only", "flat_reduce"):
            h = jnp.tile(s, (N_DEV, 1))
            xf = None
            for wout_ref, win_ref in ((wout0_ref, win1_ref),
                                      (wout1_ref, win2_ref),
                                      (wout2_ref, None)):
                xf = jnp.dot(h.astype(jnp.bfloat16),
                             wout_ref[...].astype(jnp.bfloat16),
                             preferred_element_type=jnp.float32)
                if win_ref is not None:
                    p = jnp.dot(xf.astype(jnp.bfloat16),
                                win_ref[...].astype(jnp.bfloat16),
                                preferred_element_type=jnp.float32)
                    h = jnp.maximum(p, 0.0)
            out_ref[...] = xf
            return

        for l, (wout_ref, wnext_ref) in enumerate(((wout0_ref, win1_ref),
                                                   (wout1_ref, win2_ref))):
            with jax.named_scope(f"ag_send#l={l}"):
                recv2[pl.ds(0, 1)] = s[None]
                ag_sends = ag_send(l)
            wob = wout_ref[...].astype(jnp.bfloat16)
            wnx = wnext_ref[...].astype(jnp.bfloat16)
            with jax.named_scope(f"own_chunk#l={l}"):
                y0 = jnp.dot(s, wob, preferred_element_type=jnp.float32)
                p0 = jnp.dot(y0.astype(jnp.bfloat16), wnx,
                             preferred_element_type=jnp.float32)
                recv1[pl.ds(0, 1)] = p0.astype(jnp.bfloat16)[None]
            if _VARIANT == "serial_ag":
                for lo, n in GROUPS:
                    ag_wait_group(lo, n)
                for d in ag_sends:
                    d.wait_send()
                ag_sends = []
            rs_sends = []
            for lo, n in GROUPS:
                with jax.named_scope(f"grp{lo}#l={l}"):
                    if _VARIANT != "serial_ag":
                        ag_wait_group(lo, n)
                    hg = recv2[pl.ds(lo, n)].reshape(n * chunk, h_dim)
                    yg = jnp.dot(hg, wob, preferred_element_type=jnp.float32)
                    pg = jnp.dot(yg.astype(jnp.bfloat16), wnx,
                                 preferred_element_type=jnp.float32)
                    qbuf[pl.ds(lo * chunk, n * chunk), :] = pg.astype(
                        jnp.bfloat16)
                    for off in range(lo, lo + n):
                        tgt = lax.rem(me + off, N_DEV)
                        woff = lax.rem(me - tgt + N_DEV, N_DEV)
                        d = pltpu.make_async_remote_copy(
                            src_ref=qbuf.at[pl.ds(off * chunk, chunk), :],
                            dst_ref=recv1.at[woff],
                            send_sem=ss1.at[off - 1],
                            recv_sem=rs1.at[woff],
                            device_id=(tgt,),
                            device_id_type=pl.DeviceIdType.MESH,
                        )
                        d.start()
                        rs_sends.append(d)
            s = reduce_relu(l + 1)
            for d in ag_sends:
                d.wait_send()
            for d in rs_sends:
                d.wait_send()

        with jax.named_scope("ag_send#l=2"):
            recv2[pl.ds(0, 1)] = s[None]
            ag_sends = ag_send(2)
        wob2 = wout2_ref[...].astype(jnp.bfloat16)
        with jax.named_scope("own_chunk#l=2"):
            y0 = jnp.dot(s, wob2, preferred_element_type=jnp.float32)
            out_ref[pl.ds(me * chunk, chunk), :] = y0
        for lo, n in GROUPS:
            with jax.named_scope(f"out_grp{lo}"):
                ag_wait_group(lo, n)
                hg = recv2[pl.ds(lo, n)].reshape(n * chunk, h_dim)
                yg = jnp.dot(hg, wob2, preferred_element_type=jnp.float32)
                for k, off in enumerate(range(lo, lo + n)):
                    row = lax.rem(me + off, N_DEV) * chunk
                    out_ref[pl.ds(row, chunk), :] = yg[
                        k * chunk:(k + 1) * chunk]
        for d in ag_sends:
            d.wait_send()

    return pl.pallas_call(
        body,
        out_shape=jax.ShapeDtypeStruct((b, d_per), jnp.float32),
        in_specs=[pl.BlockSpec(memory_space=pltpu.VMEM)] * 7,
        out_specs=pl.BlockSpec(memory_space=pltpu.VMEM),
        scratch_shapes=[
            pltpu.VMEM((b, h_dim), jnp.bfloat16),
            pltpu.VMEM((b, h_dim), jnp.bfloat16),
            pltpu.VMEM((N_DEV, chunk, h_dim), jnp.bfloat16),
            pltpu.VMEM((N_DEV, chunk, h_dim), jnp.bfloat16),
            pltpu.SemaphoreType.DMA((N_DEV - 1,)),
            pltpu.SemaphoreType.DMA((N_DEV,)),
            pltpu.SemaphoreType.DMA((N_DEV - 1,)),
            pltpu.SemaphoreType.DMA((N_DEV,)),
        ],
    )(x, Win0, Wout0, Win1, Wout1, Win2, Wout2)


# device time: 69319 ns/iter; 1.3467x vs baseline; 1.0470x over previous
import os

import jax
import jax.numpy as jnp
from jax import lax
from jax.experimental import pallas as pl
from jax.experimental.pallas import tpu as pltpu

N_DEV = 32
GROUPS = [(1, 8), (9, 8), (17, 8), (25, 7)]
_VARIANT = os.environ.get("KERNEL_VARIANT", "full")


def kernel(x, Win0, Wout0, Win1, Wout1, Win2, Wout2):
    b, d_per = x.shape
    _, h_dim = Win0.shape
    chunk = b // N_DEV

    def body(x_ref, win0_ref, wout0_ref, win1_ref, wout1_ref, win2_ref,
             wout2_ref, out_ref, pbuf, qbuf, recv1, recv2, ss1, rs1, ss2, rs2):
        me = lax.axis_index("i")

        def rs_send(src_slice_fn, l):
            sends = []
            for off in range(1, N_DEV):
                tgt = lax.rem(me + off, N_DEV)
                woff = lax.rem(me - tgt + N_DEV, N_DEV)
                d = pltpu.make_async_remote_copy(
                    src_ref=src_slice_fn(off, tgt),
                    dst_ref=recv1.at[woff],
                    send_sem=ss1.at[off - 1],
                    recv_sem=rs1.at[woff],
                    device_id=(tgt,),
                    device_id_type=pl.DeviceIdType.MESH,
                )
                d.start()
                sends.append(d)
            return sends

        def reduce_relu(l):
            with jax.named_scope(f"reduce#l={l}"):
                if True:
                    for off in range(1, N_DEV):
                        pltpu.make_async_remote_copy(
                            src_ref=recv1.at[0],
                            dst_ref=recv1.at[off],
                            send_sem=ss1.at[0],
                            recv_sem=rs1.at[off],
                            device_id=(me,),
                            device_id_type=pl.DeviceIdType.MESH,
                        ).wait_recv()
                    acc = jnp.sum(recv1[...].astype(jnp.float32), axis=0)
                    return jnp.maximum(acc, 0.0).astype(jnp.bfloat16)
                acc = recv1[pl.ds(0, 1)].astype(jnp.float32)[0]
                for lo, n in GROUPS:
                    for off in range(lo, lo + n):
                        pltpu.make_async_remote_copy(
                            src_ref=recv1.at[0],
                            dst_ref=recv1.at[off],
                            send_sem=ss1.at[0],
                            recv_sem=rs1.at[off],
                            device_id=(me,),
                            device_id_type=pl.DeviceIdType.MESH,
                        ).wait_recv()
                    acc = acc + jnp.sum(
                        recv1[pl.ds(lo, n)].astype(jnp.float32), axis=0)
                return jnp.maximum(acc, 0.0).astype(jnp.bfloat16)

        def ag_send(l):
            sends = []
            for off in range(1, N_DEV):
                tgt = lax.rem(me + off, N_DEV)
                woff = lax.rem(me - tgt + N_DEV, N_DEV)
                d = pltpu.make_async_remote_copy(
                    src_ref=recv2.at[0],
                    dst_ref=recv2.at[woff],
                    send_sem=ss2.at[off - 1],
                    recv_sem=rs2.at[woff],
                    device_id=(tgt,),
                    device_id_type=pl.DeviceIdType.MESH,
                )
                d.start()
                sends.append(d)
            return sends

        def ag_wait_group(lo, n):
            for off in reversed(range(lo, lo + n)):
                pltpu.make_async_remote_copy(
                    src_ref=recv2.at[0],
                    dst_ref=recv2.at[off],
                    send_sem=ss2.at[0],
                    recv_sem=rs2.at[off],
                    device_id=(me,),
                    device_id_type=pl.DeviceIdType.MESH,
                ).wait_recv()

        if _VARIANT == "nocomm":
            xb = x_ref[...].astype(jnp.bfloat16)
            xf = None
            for win_ref, wout_ref in ((win0_ref, wout0_ref),
                                      (win1_ref, wout1_ref),
                                      (win2_ref, wout2_ref)):
                p = jnp.dot(xb, win_ref[...].astype(jnp.bfloat16),
                            preferred_element_type=jnp.float32)
                h = jnp.maximum(p, 0.0).astype(jnp.bfloat16)
                xf = jnp.dot(h, wout_ref[...].astype(jnp.bfloat16),
                             preferred_element_type=jnp.float32)
                xb = xf.astype(jnp.bfloat16)
            out_ref[...] = xf
            return

        with jax.named_scope("matmul_in#l=0"):
            xb = x_ref[...].astype(jnp.bfloat16)
            partial = jnp.dot(xb, win0_ref[...].astype(jnp.bfloat16),
                              preferred_element_type=jnp.float32)
            pbuf[...] = partial.astype(jnp.bfloat16)
        with jax.named_scope("rs_send#l=0"):
            rs_sends = rs_send(
                lambda off, tgt: pbuf.at[pl.ds(tgt * chunk, chunk), :], 0)
            recv1[pl.ds(0, 1)] = pbuf[pl.ds(me * chunk, chunk), :][None]
        s = reduce_relu(0)
        for d in rs_sends:
            d.wait_send()

        if _VARIANT in ("rs0_only", "flat_reduce"):
            h = jnp.tile(s, (N_DEV, 1))
            xf = None
            for wout_ref, win_ref in ((wout0_ref, win1_ref),
                                      (wout1_ref, win2_ref),
                                      (wout2_ref, None)):
                xf = jnp.dot(h.astype(jnp.bfloat16),
                             wout_ref[...].astype(jnp.bfloat16),
                             preferred_element_type=jnp.float32)
                if win_ref is not None:
                    p = jnp.dot(xf.astype(jnp.bfloat16),
                                win_ref[...].astype(jnp.bfloat16),
                                preferred_element_type=jnp.float32)
                    h = jnp.maximum(p, 0.0)
            out_ref[...] = xf
            return

        for l, (wout_ref, wnext_ref) in enumerate(((wout0_ref, win1_ref),
                                                   (wout1_ref, win2_ref))):
            with jax.named_scope(f"ag_send#l={l}"):
                recv2[pl.ds(0, 1)] = s[None]
                ag_sends = ag_send(l)
            wob = wout_ref[...].astype(jnp.bfloat16)
            wnx = wnext_ref[...].astype(jnp.bfloat16)
            with jax.named_scope(f"own_chunk#l={l}"):
                y0 = jnp.dot(s, wob, preferred_element_type=jnp.float32)
                p0 = jnp.dot(y0.astype(jnp.bfloat16), wnx,
                             preferred_element_type=jnp.float32)
                recv1[pl.ds(0, 1)] = p0.astype(jnp.bfloat16)[None]
            if _VARIANT == "serial_ag":
                for lo, n in GROUPS:
                    ag_wait_group(lo, n)
                for d in ag_sends:
                    d.wait_send()
                ag_sends = []
            rs_sends = []
            for lo, n in reversed(GROUPS):
                with jax.named_scope(f"grp{lo}#l={l}"):
                    if _VARIANT != "serial_ag":
                        ag_wait_group(lo, n)
                    hg = recv2[pl.ds(lo, n)].reshape(n * chunk, h_dim)
                    yg = jnp.dot(hg, wob, preferred_element_type=jnp.float32)
                    pg = jnp.dot(yg.astype(jnp.bfloat16), wnx,
                                 preferred_element_type=jnp.float32)
                    qbuf[pl.ds(lo * chunk, n * chunk), :] = pg.astype(
                        jnp.bfloat16)
                    for off in range(lo, lo + n):
                        tgt = lax.rem(me + off, N_DEV)
                        woff = lax.rem(me - tgt + N_DEV, N_DEV)
                        d = pltpu.make_async_remote_copy(
                            src_ref=qbuf.at[pl.ds(off * chunk, chunk), :],
                            dst_ref=recv1.at[woff],
                            send_sem=ss1.at[off - 1],
                            recv_sem=rs1.at[woff],
                            device_id=(tgt,),
                            device_id_type=pl.DeviceIdType.MESH,
                        )
                        d.start()
                        rs_sends.append(d)
            s = reduce_relu(l + 1)
            for d in ag_sends:
                d.wait_send()
            for d in rs_sends:
                d.wait_send()

        with jax.named_scope("ag_send#l=2"):
            recv2[pl.ds(0, 1)] = s[None]
            ag_sends = ag_send(2)
        wob2 = wout2_ref[...].astype(jnp.bfloat16)
        with jax.named_scope("own_chunk#l=2"):
            y0 = jnp.dot(s, wob2, preferred_element_type=jnp.float32)
            out_ref[pl.ds(me * chunk, chunk), :] = y0
        for lo, n in reversed(GROUPS):
            with jax.named_scope(f"out_grp{lo}"):
                ag_wait_group(lo, n)
                hg = recv2[pl.ds(lo, n)].reshape(n * chunk, h_dim)
                yg = jnp.dot(hg, wob2, preferred_element_type=jnp.float32)
                for k, off in enumerate(range(lo, lo + n)):
                    row = lax.rem(me + off, N_DEV) * chunk
                    out_ref[pl.ds(row, chunk), :] = yg[
                        k * chunk:(k + 1) * chunk]
        for d in ag_sends:
            d.wait_send()

    return pl.pallas_call(
        body,
        out_shape=jax.ShapeDtypeStruct((b, d_per), jnp.float32),
        in_specs=[pl.BlockSpec(memory_space=pltpu.VMEM)] * 7,
        out_specs=pl.BlockSpec(memory_space=pltpu.VMEM),
        scratch_shapes=[
            pltpu.VMEM((b, h_dim), jnp.bfloat16),
            pltpu.VMEM((b, h_dim), jnp.bfloat16),
            pltpu.VMEM((N_DEV, chunk, h_dim), jnp.bfloat16),
            pltpu.VMEM((N_DEV, chunk, h_dim), jnp.bfloat16),
            pltpu.SemaphoreType.DMA((N_DEV - 1,)),
            pltpu.SemaphoreType.DMA((N_DEV,)),
            pltpu.SemaphoreType.DMA((N_DEV - 1,)),
            pltpu.SemaphoreType.DMA((N_DEV,)),
        ],
    )(x, Win0, Wout0, Win1, Wout1, Win2, Wout2)
